# baseline (speedup 1.0000x reference)
import functools
import jax
import jax.numpy as jnp
from jax import lax
from jax.experimental import pallas as pl
from jax.experimental.pallas import tpu as pltpu, tpu_sc as plsc

K = 3
BM = 1024


def _topk_body(ps_ref, posT_ref, idx_ref, w_ref):
    n = posT_ref.shape[1]
    bm = ps_ref.shape[1]

    pn = jnp.sum(posT_ref[...] * posT_ref[...], axis=0, keepdims=True)
    e = pn
    qn = jnp.zeros((bm, 1), dtype=jnp.float32)
    for c in range(3):
        q_c = ps_ref[c, :].reshape(bm, 1)
        p_c = posT_ref[c, :].reshape(1, n)
        e = e + q_c * (-2.0 * p_c)
        qn = qn + q_c * q_c

    iota = lax.broadcasted_iota(jnp.int32, (bm, n), 1)
    wsum = jnp.zeros((bm, 1), dtype=jnp.float32)
    ws = []
    for k in range(K):
        m_e = jnp.min(e, axis=1, keepdims=True)
        w_k = 1.0 / jnp.maximum(m_e + qn, 1e-16)
        hit = e == m_e
        i_k = jnp.min(jnp.where(hit, iota, n), axis=1, keepdims=True)
        idx_ref[:, k:k + 1] = i_k
        if k < K - 1:
            e = jnp.where(hit, jnp.inf, e)
        ws.append(w_k)
        wsum = wsum + w_k
    for k in range(K):
        w_ref[:, k:k + 1] = ws[k] / wsum


def _topk(pos_skipT, posT, m):
    n = posT.shape[1]
    return pl.pallas_call(
        _topk_body,
        grid=(m // BM,),
        in_specs=[
            pl.BlockSpec((3, BM), lambda i: (0, i)),
            pl.BlockSpec((3, n), lambda i: (0, 0)),
        ],
        out_specs=[
            pl.BlockSpec((BM, K), lambda i: (i, 0)),
            pl.BlockSpec((BM, K), lambda i: (i, 0)),
        ],
        out_shape=[
            jax.ShapeDtypeStruct((m, K), jnp.int32),
            jax.ShapeDtypeStruct((m, K), jnp.float32),
        ],
    )(pos_skipT, posT)


def _make_sc_gather(v, d_feat, b):
    info = plsc.get_sparse_core_info()
    nw = info.num_cores * info.num_subcores
    b_per_w = b // nw
    chunk = 192
    nchunks = b_per_w // chunk
    mesh = plsc.VectorSubcoreMesh(core_axis_name="c", subcore_axis_name="s")

    @functools.partial(
        pl.kernel, mesh=mesh,
        out_type=jax.ShapeDtypeStruct((b, d_feat), jnp.float32),
        scratch_types=[
            pltpu.VMEM((chunk,), jnp.int32),
            pltpu.VMEM((chunk, d_feat), jnp.float32),
            pltpu.SemaphoreType.DMA,
        ],
    )
    def gather_kernel(table_hbm, idx_hbm, out_hbm, idx_v, rows_v, sem):
        wid = lax.axis_index("s") * info.num_cores + lax.axis_index("c")
        base = wid * b_per_w

        def body(ci, _):
            off = base + ci * chunk
            pltpu.sync_copy(idx_hbm.at[pl.ds(off, chunk)], idx_v)
            pltpu.async_copy(table_hbm.at[idx_v], rows_v, sem).wait()
            pltpu.sync_copy(rows_v, out_hbm.at[pl.ds(off, chunk)])
            return ()

        lax.fori_loop(0, nchunks, body, (), unroll=False)

    return gather_kernel


def _combine_body(f_ref, w_ref, w1_ref, b1_ref, w2_ref, b2_ref, out_ref):
    interp = (f_ref[:, 0, :] * w_ref[:, 0:1]
              + f_ref[:, 1, :] * w_ref[:, 1:2]
              + f_ref[:, 2, :] * w_ref[:, 2:3])
    h1 = jnp.dot(interp, w1_ref[...], preferred_element_type=jnp.float32)
    h1 = jnp.maximum(h1 + b1_ref[...], 0.0)
    h2 = jnp.dot(h1, w2_ref[...], preferred_element_type=jnp.float32)
    out_ref[...] = h2 + b2_ref[...]


def _combine(feats, w, W1, b1, W2, b2, m, d_feat, h_feat):
    return pl.pallas_call(
        _combine_body,
        grid=(m // BM,),
        in_specs=[
            pl.BlockSpec((BM, K, d_feat), lambda i: (i, 0, 0)),
            pl.BlockSpec((BM, K), lambda i: (i, 0)),
            pl.BlockSpec((d_feat, h_feat), lambda i: (0, 0)),
            pl.BlockSpec((1, h_feat), lambda i: (0, 0)),
            pl.BlockSpec((h_feat, h_feat), lambda i: (0, 0)),
            pl.BlockSpec((1, h_feat), lambda i: (0, 0)),
        ],
        out_specs=pl.BlockSpec((BM, h_feat), lambda i: (i, 0)),
        out_shape=jax.ShapeDtypeStruct((m, h_feat), jnp.float32),
    )(feats, w, W1, b1.reshape(1, -1), W2, b2.reshape(1, -1))


def kernel(x, pos, x_skip, pos_skip, assign_index, W1, b1, W2, b2):
    del x_skip, assign_index
    n, d_feat = x.shape
    m = pos_skip.shape[0]
    h_feat = W2.shape[1]

    posT = pos.T
    pos_skipT = pos_skip.T

    idx, w = _topk(pos_skipT, posT, m)
    idx_flat = idx.reshape(m * K)

    feats_flat = _make_sc_gather(n, d_feat, m * K)(x, idx_flat)
    feats = feats_flat.reshape(m, K, d_feat)

    out = _combine(feats, w, W1, b1, W2, b2, m, d_feat, h_feat)
    return (out, pos_skip)


# speedup vs baseline: 2.4320x; 2.4320x over previous
import jax
import jax.numpy as jnp
from jax.experimental import pallas as pl

K = 3
BM = 1024


def _fused_body(ps_ref, posT_ref, x_ref, w1_ref, b1_ref, w2_ref, b2_ref,
                out_ref):
    n = posT_ref.shape[1]
    bm = ps_ref.shape[1]

    pn = jnp.sum(posT_ref[...] * posT_ref[...], axis=0, keepdims=True)
    e = pn
    qn = jnp.zeros((bm, 1), dtype=jnp.float32)
    for c in range(3):
        q_c = ps_ref[c, :].reshape(bm, 1)
        p_c = posT_ref[c, :].reshape(1, n)
        e = e + q_c * (-2.0 * p_c)
        qn = qn + q_c * q_c

    s = jnp.zeros((bm, n), dtype=jnp.float32)
    wsum = jnp.zeros((bm, 1), dtype=jnp.float32)
    for k in range(K):
        m_e = jnp.min(e, axis=1, keepdims=True)
        w_k = 1.0 / jnp.maximum(m_e + qn, 1e-16)
        hit = e == m_e
        s = jnp.where(hit, w_k, s)
        if k < K - 1:
            e = jnp.where(hit, jnp.inf, e)
        wsum = wsum + w_k

    interp = jnp.dot(s, x_ref[...], preferred_element_type=jnp.float32)
    interp = interp / wsum

    h1 = jnp.dot(interp, w1_ref[...], preferred_element_type=jnp.float32)
    h1 = jnp.maximum(h1 + b1_ref[...], 0.0)
    h2 = jnp.dot(h1, w2_ref[...], preferred_element_type=jnp.float32)
    out_ref[...] = h2 + b2_ref[...]


def kernel(x, pos, x_skip, pos_skip, assign_index, W1, b1, W2, b2):
    del x_skip, assign_index
    n, d_feat = x.shape
    m = pos_skip.shape[0]
    h_feat = W2.shape[1]

    posT = pos.T
    pos_skipT = pos_skip.T
    x_bf = x.astype(jnp.bfloat16)
    w1_bf = W1.astype(jnp.bfloat16)
    w2_bf = W2.astype(jnp.bfloat16)
    b1_2d = b1.reshape(1, -1)
    b2_2d = b2.reshape(1, -1)

    grid = (m // BM,)
    out = pl.pallas_call(
        _fused_body,
        grid=grid,
        in_specs=[
            pl.BlockSpec((3, BM), lambda i: (0, i)),
            pl.BlockSpec((3, n), lambda i: (0, 0)),
            pl.BlockSpec((n, d_feat), lambda i: (0, 0)),
            pl.BlockSpec((d_feat, h_feat), lambda i: (0, 0)),
            pl.BlockSpec((1, h_feat), lambda i: (0, 0)),
            pl.BlockSpec((h_feat, h_feat), lambda i: (0, 0)),
            pl.BlockSpec((1, h_feat), lambda i: (0, 0)),
        ],
        out_specs=pl.BlockSpec((BM, h_feat), lambda i: (i, 0)),
        out_shape=jax.ShapeDtypeStruct((m, h_feat), jnp.float32),
    )(pos_skipT, posT, x_bf, w1_bf, b1_2d, w2_bf, b2_2d)

    return (out, pos_skip)
